# baseline (speedup 1.0000x reference)
import jax
import jax.numpy as jnp
from jax.experimental import pallas as pl
from jax.experimental.pallas import tpu as pltpu

_TR = 80
_B = 10


def _make_body(n, nfeat, nhid, nclass):
    nt = n // _TR
    total = 2 * nt

    def _tile_row(t):
        return jax.lax.rem(t, nt) * _TR

    def _adj_copy(adj_ref, ring_ref, sem_ref, t, slot):
        row = _tile_row(t)
        return pltpu.make_async_copy(
            adj_ref.at[pl.ds(row, _TR), :], ring_ref.at[slot], sem_ref.at[slot])

    def _out_copy(stage_ref, out_ref, osem_ref, t, slot):
        return pltpu.make_async_copy(
            stage_ref.at[slot], out_ref.at[pl.ds(_tile_row(t), _TR), :],
            osem_ref.at[slot])

    def _body(x_ref, adj_ref, w1_ref, b1_ref, w2_ref, b2_ref, out_ref,
              ring_ref, xbuf_ref, stage_ref, s_ref, g_ref,
              sem_ref, xsem_ref, osem_ref):
        xcopy = pltpu.make_async_copy(x_ref, xbuf_ref, xsem_ref)
        xcopy.start()
        for j in range(_B - 1):
            _adj_copy(adj_ref, ring_ref, sem_ref, j, j).start()

        xcopy.wait()
        s_ref[...] = jnp.dot(xbuf_ref[...], w1_ref[...],
                             preferred_element_type=jnp.float32)

        def outer(step, _):
            for j in range(_B):
                t = step * _B + j
                row = _tile_row(t)
                _adj_copy(adj_ref, ring_ref, sem_ref, t, j).wait()

                @pl.when(t < nt)
                def _sweep1():
                    h = jnp.dot(ring_ref[j], s_ref[...],
                                preferred_element_type=jnp.float32)
                    h = jnp.maximum(h + b1_ref[...], 0.0)
                    g_ref[pl.ds(row, _TR), :] = jnp.dot(
                        h, w2_ref[...], preferred_element_type=jnp.float32)

                @pl.when(t >= nt)
                def _sweep2():
                    slot = jax.lax.rem(t - nt, 2)

                    @pl.when(t >= nt + 2)
                    def _reclaim():
                        _out_copy(stage_ref, out_ref, osem_ref, t - 2,
                                  slot).wait()

                    v = jnp.dot(ring_ref[j], g_ref[...],
                                preferred_element_type=jnp.float32)
                    v = v + b2_ref[...]
                    m = jnp.max(v, axis=1, keepdims=True)
                    lse = jnp.log(jnp.sum(jnp.exp(v - m), axis=1,
                                          keepdims=True)) + m
                    stage_ref[slot] = v - lse
                    _out_copy(stage_ref, out_ref, osem_ref, t, slot).start()

                @pl.when(t + _B - 1 < total)
                def _refill():
                    _adj_copy(adj_ref, ring_ref, sem_ref, t + _B - 1,
                              (j + _B - 1) % _B).start()
            return 0

        jax.lax.fori_loop(0, total // _B, outer, 0)

        _out_copy(stage_ref, out_ref, osem_ref, total - 2, (nt - 2) % 2).wait()
        _out_copy(stage_ref, out_ref, osem_ref, total - 1, (nt - 1) % 2).wait()

    return _body


def kernel(x, adj, W1, b1, W2, b2):
    n, nfeat = x.shape
    nhid = W1.shape[1]
    nclass = W2.shape[1]
    b1r = b1.reshape(1, nhid)
    b2r = b2.reshape(1, nclass)

    hbm = pl.BlockSpec(memory_space=pltpu.MemorySpace.HBM)
    vmem = pl.BlockSpec(memory_space=pltpu.MemorySpace.VMEM)

    out = pl.pallas_call(
        _make_body(n, nfeat, nhid, nclass),
        in_specs=[hbm, hbm, vmem, vmem, vmem, vmem],
        out_specs=hbm,
        out_shape=jax.ShapeDtypeStruct((n, nclass), jnp.float32),
        scratch_shapes=[
            pltpu.VMEM((_B, _TR, n), jnp.float32),
            pltpu.VMEM((n, nfeat), jnp.float32),
            pltpu.VMEM((2, _TR, nclass), jnp.float32),
            pltpu.VMEM((n, nhid), jnp.float32),
            pltpu.VMEM((n, nclass), jnp.float32),
            pltpu.SemaphoreType.DMA((_B,)),
            pltpu.SemaphoreType.DMA,
            pltpu.SemaphoreType.DMA((2,)),
        ],
        compiler_params=pltpu.CompilerParams(
            vmem_limit_bytes=64 * 1024 * 1024),
    )(x, adj, W1, b1r, W2, b2r)

    return out


# speedup vs baseline: 1.0121x; 1.0121x over previous
import jax
import jax.numpy as jnp
from jax.experimental import pallas as pl
from jax.experimental.pallas import tpu as pltpu

_TR = 200
_B = 4


def _make_body(n, nfeat, nhid, nclass):
    nt = n // _TR
    total = 2 * nt

    def _tile_row(t):
        return jax.lax.rem(t, nt) * _TR

    def _adj_copy(adj_ref, ring_ref, sem_ref, t, slot):
        row = _tile_row(t)
        return pltpu.make_async_copy(
            adj_ref.at[pl.ds(row, _TR), :], ring_ref.at[slot], sem_ref.at[slot])

    def _out_copy(stage_ref, out_ref, osem_ref, t, slot):
        return pltpu.make_async_copy(
            stage_ref.at[slot], out_ref.at[pl.ds(_tile_row(t), _TR), :],
            osem_ref.at[slot])

    def _body(x_ref, adj_ref, w1_ref, b1_ref, w2_ref, b2_ref, out_ref,
              ring_ref, xbuf_ref, stage_ref, s_ref, g_ref,
              sem_ref, xsem_ref, osem_ref):
        xcopy = pltpu.make_async_copy(x_ref, xbuf_ref, xsem_ref)
        xcopy.start()
        for j in range(_B - 1):
            _adj_copy(adj_ref, ring_ref, sem_ref, j, j).start()

        xcopy.wait()
        s_ref[...] = jnp.dot(xbuf_ref[...], w1_ref[...],
                             preferred_element_type=jnp.float32)

        def outer(step, _):
            for j in range(_B):
                t = step * _B + j
                row = _tile_row(t)
                _adj_copy(adj_ref, ring_ref, sem_ref, t, j).wait()

                @pl.when(t < nt)
                def _sweep1():
                    h = jnp.dot(ring_ref[j], s_ref[...],
                                preferred_element_type=jnp.float32)
                    h = jnp.maximum(h + b1_ref[...], 0.0)
                    g_ref[pl.ds(row, _TR), :] = jnp.dot(
                        h, w2_ref[...], preferred_element_type=jnp.float32)

                @pl.when(t >= nt)
                def _sweep2():
                    slot = jax.lax.rem(t - nt, 2)

                    @pl.when(t >= nt + 2)
                    def _reclaim():
                        _out_copy(stage_ref, out_ref, osem_ref, t - 2,
                                  slot).wait()

                    v = jnp.dot(ring_ref[j], g_ref[...],
                                preferred_element_type=jnp.float32)
                    v = v + b2_ref[...]
                    m = jnp.max(v, axis=1, keepdims=True)
                    lse = jnp.log(jnp.sum(jnp.exp(v - m), axis=1,
                                          keepdims=True)) + m
                    stage_ref[slot] = v - lse
                    _out_copy(stage_ref, out_ref, osem_ref, t, slot).start()

                @pl.when(t + _B - 1 < total)
                def _refill():
                    _adj_copy(adj_ref, ring_ref, sem_ref, t + _B - 1,
                              (j + _B - 1) % _B).start()
            return 0

        jax.lax.fori_loop(0, total // _B, outer, 0)

        _out_copy(stage_ref, out_ref, osem_ref, total - 2, (nt - 2) % 2).wait()
        _out_copy(stage_ref, out_ref, osem_ref, total - 1, (nt - 1) % 2).wait()

    return _body


def kernel(x, adj, W1, b1, W2, b2):
    n, nfeat = x.shape
    nhid = W1.shape[1]
    nclass = W2.shape[1]
    b1r = b1.reshape(1, nhid)
    b2r = b2.reshape(1, nclass)

    hbm = pl.BlockSpec(memory_space=pltpu.MemorySpace.HBM)
    vmem = pl.BlockSpec(memory_space=pltpu.MemorySpace.VMEM)

    out = pl.pallas_call(
        _make_body(n, nfeat, nhid, nclass),
        in_specs=[hbm, hbm, vmem, vmem, vmem, vmem],
        out_specs=hbm,
        out_shape=jax.ShapeDtypeStruct((n, nclass), jnp.float32),
        scratch_shapes=[
            pltpu.VMEM((_B, _TR, n), jnp.float32),
            pltpu.VMEM((n, nfeat), jnp.float32),
            pltpu.VMEM((2, _TR, nclass), jnp.float32),
            pltpu.VMEM((n, nhid), jnp.float32),
            pltpu.VMEM((n, nclass), jnp.float32),
            pltpu.SemaphoreType.DMA((_B,)),
            pltpu.SemaphoreType.DMA,
            pltpu.SemaphoreType.DMA((2,)),
        ],
        compiler_params=pltpu.CompilerParams(
            vmem_limit_bytes=64 * 1024 * 1024),
    )(x, adj, W1, b1r, W2, b2r)

    return out


# speedup vs baseline: 1.0158x; 1.0037x over previous
import jax
import jax.numpy as jnp
from jax.experimental import pallas as pl
from jax.experimental.pallas import tpu as pltpu

_TR = 200
_B = 4


def _make_body(n, nfeat, nhid, nclass):
    nt = n // _TR
    total = 2 * nt

    def _tile_row(t):
        return jax.lax.rem(t, nt) * _TR

    def _adj_copy(adj_ref, ring_ref, sem_ref, t, slot):
        row = _tile_row(t)
        return pltpu.make_async_copy(
            adj_ref.at[pl.ds(row, _TR), :], ring_ref.at[slot], sem_ref.at[slot])

    def _out_copy(stage_ref, out_ref, osem_ref, t, slot):
        return pltpu.make_async_copy(
            stage_ref.at[slot], out_ref.at[pl.ds(_tile_row(t), _TR), :],
            osem_ref.at[slot])

    def _body(x_ref, adj_ref, w1_ref, b1_ref, w2_ref, b2_ref, out_ref,
              ring_ref, xbuf_ref, stage_ref, s_ref, g_ref,
              sem_ref, xsem_ref, osem_ref):
        xcopy = pltpu.make_async_copy(x_ref, xbuf_ref, xsem_ref)
        xcopy.start()
        for j in range(_B - 1):
            _adj_copy(adj_ref, ring_ref, sem_ref, j, j).start()

        xcopy.wait()
        s_ref[...] = jnp.dot(xbuf_ref[...], w1_ref[...],
                             preferred_element_type=jnp.float32)

        def outer(step, _):
            for j in range(_B):
                t = step * _B + j
                row = _tile_row(t)
                _adj_copy(adj_ref, ring_ref, sem_ref, t, j).wait()

                @pl.when(t + _B - 1 < total)
                def _refill():
                    _adj_copy(adj_ref, ring_ref, sem_ref, t + _B - 1,
                              (j + _B - 1) % _B).start()

                @pl.when(t < nt)
                def _sweep1():
                    h = jnp.dot(ring_ref[j], s_ref[...],
                                preferred_element_type=jnp.float32)
                    h = jnp.maximum(h + b1_ref[...], 0.0)
                    g_ref[pl.ds(row, _TR), :] = jnp.dot(
                        h, w2_ref[...], preferred_element_type=jnp.float32)

                @pl.when(t >= nt)
                def _sweep2():
                    slot = jax.lax.rem(t - nt, 2)

                    @pl.when(t >= nt + 2)
                    def _reclaim():
                        _out_copy(stage_ref, out_ref, osem_ref, t - 2,
                                  slot).wait()

                    v = jnp.dot(ring_ref[j], g_ref[...],
                                preferred_element_type=jnp.float32)
                    v = v + b2_ref[...]
                    m = jnp.max(v, axis=1, keepdims=True)
                    lse = jnp.log(jnp.sum(jnp.exp(v - m), axis=1,
                                          keepdims=True)) + m
                    stage_ref[slot] = v - lse
                    _out_copy(stage_ref, out_ref, osem_ref, t, slot).start()

            return 0

        jax.lax.fori_loop(0, total // _B, outer, 0)

        _out_copy(stage_ref, out_ref, osem_ref, total - 2, (nt - 2) % 2).wait()
        _out_copy(stage_ref, out_ref, osem_ref, total - 1, (nt - 1) % 2).wait()

    return _body


def kernel(x, adj, W1, b1, W2, b2):
    n, nfeat = x.shape
    nhid = W1.shape[1]
    nclass = W2.shape[1]
    b1r = b1.reshape(1, nhid)
    b2r = b2.reshape(1, nclass)

    hbm = pl.BlockSpec(memory_space=pltpu.MemorySpace.HBM)
    vmem = pl.BlockSpec(memory_space=pltpu.MemorySpace.VMEM)

    out = pl.pallas_call(
        _make_body(n, nfeat, nhid, nclass),
        in_specs=[hbm, hbm, vmem, vmem, vmem, vmem],
        out_specs=hbm,
        out_shape=jax.ShapeDtypeStruct((n, nclass), jnp.float32),
        scratch_shapes=[
            pltpu.VMEM((_B, _TR, n), jnp.float32),
            pltpu.VMEM((n, nfeat), jnp.float32),
            pltpu.VMEM((2, _TR, nclass), jnp.float32),
            pltpu.VMEM((n, nhid), jnp.float32),
            pltpu.VMEM((n, nclass), jnp.float32),
            pltpu.SemaphoreType.DMA((_B,)),
            pltpu.SemaphoreType.DMA,
            pltpu.SemaphoreType.DMA((2,)),
        ],
        compiler_params=pltpu.CompilerParams(
            vmem_limit_bytes=64 * 1024 * 1024),
    )(x, adj, W1, b1r, W2, b2r)

    return out


# speedup vs baseline: 1.0179x; 1.0021x over previous
import jax
import jax.numpy as jnp
from jax.experimental import pallas as pl
from jax.experimental.pallas import tpu as pltpu

_TR = 200
_B = 4


def _make_body(n, nfeat, nhid, nclass):
    nt = n // _TR
    total = 2 * nt

    def _tile_row(t):
        return jax.lax.rem(t, nt) * _TR

    def _adj_copy(adj_ref, ring_ref, sem_ref, t, slot):
        row = _tile_row(t)
        return pltpu.make_async_copy(
            adj_ref.at[pl.ds(row, _TR), :], ring_ref.at[slot], sem_ref.at[slot])

    def _out_copy(stage_ref, out_ref, osem_ref, t, slot):
        return pltpu.make_async_copy(
            stage_ref.at[slot], out_ref.at[pl.ds(_tile_row(t), _TR), :],
            osem_ref.at[slot])

    def _body(x_ref, adj_ref, w1_ref, b1_ref, w2_ref, b2_ref, out_ref,
              ring_ref, xbuf_ref, stage_ref, s_ref, g_ref,
              sem_ref, xsem_ref, osem_ref):
        xh = n // 5
        xcopies = [
            pltpu.make_async_copy(
                x_ref.at[pl.ds(c * xh, xh), :],
                xbuf_ref.at[pl.ds(c * xh, xh), :], xsem_ref)
            for c in range(5)
        ]
        for c in xcopies:
            c.start()
        for j in range(_B - 1):
            _adj_copy(adj_ref, ring_ref, sem_ref, j, j).start()

        for c in xcopies:
            c.wait()
        s_ref[...] = jnp.dot(xbuf_ref[...], w1_ref[...],
                             preferred_element_type=jnp.float32)

        def outer(step, _):
            for j in range(_B):
                t = step * _B + j
                row = _tile_row(t)
                _adj_copy(adj_ref, ring_ref, sem_ref, t, j).wait()

                @pl.when(t + _B - 1 < total)
                def _refill():
                    _adj_copy(adj_ref, ring_ref, sem_ref, t + _B - 1,
                              (j + _B - 1) % _B).start()

                @pl.when(t < nt)
                def _sweep1():
                    h = jnp.dot(ring_ref[j], s_ref[...],
                                preferred_element_type=jnp.float32)
                    h = jnp.maximum(h + b1_ref[...], 0.0)
                    g_ref[pl.ds(row, _TR), :] = jnp.dot(
                        h, w2_ref[...], preferred_element_type=jnp.float32)

                @pl.when(t >= nt)
                def _sweep2():
                    slot = jax.lax.rem(t - nt, 2)

                    @pl.when(t >= nt + 2)
                    def _reclaim():
                        _out_copy(stage_ref, out_ref, osem_ref, t - 2,
                                  slot).wait()

                    v = jnp.dot(ring_ref[j], g_ref[...],
                                preferred_element_type=jnp.float32)
                    v = v + b2_ref[...]
                    m = jnp.max(v, axis=1, keepdims=True)
                    lse = jnp.log(jnp.sum(jnp.exp(v - m), axis=1,
                                          keepdims=True)) + m
                    stage_ref[slot] = v - lse
                    _out_copy(stage_ref, out_ref, osem_ref, t, slot).start()

            return 0

        jax.lax.fori_loop(0, total // _B, outer, 0)

        _out_copy(stage_ref, out_ref, osem_ref, total - 2, (nt - 2) % 2).wait()
        _out_copy(stage_ref, out_ref, osem_ref, total - 1, (nt - 1) % 2).wait()

    return _body


def kernel(x, adj, W1, b1, W2, b2):
    n, nfeat = x.shape
    nhid = W1.shape[1]
    nclass = W2.shape[1]
    b1r = b1.reshape(1, nhid)
    b2r = b2.reshape(1, nclass)

    hbm = pl.BlockSpec(memory_space=pltpu.MemorySpace.HBM)
    vmem = pl.BlockSpec(memory_space=pltpu.MemorySpace.VMEM)

    out = pl.pallas_call(
        _make_body(n, nfeat, nhid, nclass),
        in_specs=[hbm, hbm, vmem, vmem, vmem, vmem],
        out_specs=hbm,
        out_shape=jax.ShapeDtypeStruct((n, nclass), jnp.float32),
        scratch_shapes=[
            pltpu.VMEM((_B, _TR, n), jnp.float32),
            pltpu.VMEM((n, nfeat), jnp.float32),
            pltpu.VMEM((2, _TR, nclass), jnp.float32),
            pltpu.VMEM((n, nhid), jnp.float32),
            pltpu.VMEM((n, nclass), jnp.float32),
            pltpu.SemaphoreType.DMA((_B,)),
            pltpu.SemaphoreType.DMA,
            pltpu.SemaphoreType.DMA((2,)),
        ],
        compiler_params=pltpu.CompilerParams(
            vmem_limit_bytes=64 * 1024 * 1024),
    )(x, adj, W1, b1r, W2, b2r)

    return out
